# baseline (speedup 1.0000x reference)
import functools

import jax
import jax.numpy as jnp
from jax import lax
from jax.experimental import pallas as pl
from jax.experimental.pallas import tpu as pltpu
from jax.experimental.pallas import tpu_sc as plsc

N = 10000
E = 320000
D = 128
H = 4
EF = 16
NET = 64

NPAD = 10240
FLAT = NPAD * H
CHUNK = 1280
NW = 32
NCHUNKS = E // CHUNK
K = -(-NCHUNKS // NW)
GROUPS = CHUNK // 16
SROWS = CHUNK * H // 128
ROWS = E * H // 128
STRIPE = FLAT // 16


def _tc_body(x_ref, wfc_ref, al_ref, ar_ref, eemb_ref, we_ref, ae_ref,
             elr_ref, ee_ref):
    al = al_ref[0]
    ar = ar_ref[0]
    cols = []
    for a in (al, ar):
        for h in range(H):
            wh = wfc_ref[:, D * h:D * (h + 1)]
            cols.append(lax.dot_general(
                wh, a[h:h + 1, :], (((1,), (1,)), ((), ())),
                preferred_element_type=jnp.float32))
    v = jnp.concatenate(cols, axis=1)
    elr_ref[...] = lax.dot_general(
        x_ref[...], v, (((1,), (0,)), ((), ())),
        preferred_element_type=jnp.float32)
    proj = lax.dot_general(
        eemb_ref[...], we_ref[...], (((1,), (0,)), ((), ())),
        preferred_element_type=jnp.float32)
    ae = ae_ref[0]
    ecols = []
    for h in range(H):
        ecols.append(lax.dot_general(
            proj[:, EF * h:EF * (h + 1)], ae[h:h + 1, :],
            (((1,), (1,)), ((), ())),
            preferred_element_type=jnp.float32))
    ee_ref[...] = jnp.concatenate(ecols, axis=1)


_tc_project = pl.pallas_call(
    _tc_body,
    out_shape=[
        jax.ShapeDtypeStruct((N, 2 * H), jnp.float32),
        jax.ShapeDtypeStruct((NET, H), jnp.float32),
    ],
)


_mesh = plsc.VectorSubcoreMesh(core_axis_name="c", subcore_axis_name="s")
_sc_params = pltpu.CompilerParams(needs_layout_passes=False)
_sc_params_dense = pltpu.CompilerParams(
    needs_layout_passes=False, use_tc_tiling_on_sc=False)


def _sc_logits_body(ei_hbm, ef_hbm, elr_hbm, ee_hbm,
                    s_hbm, dpart_hbm,
                    elr_v, ee_v,
                    srcb0, dstb0, efb0, s2d0, idx2d0,
                    srcb1, dstb1, efb1, s2d1, idx2d1,
                    zbuf, shared,
                    insem0, insem1, asem0, asem1, wsem0, wsem1):
    cid = lax.axis_index("c")
    sid = lax.axis_index("s")
    w = sid * 2 + cid

    bufs = (
        (srcb0, dstb0, efb0, s2d0, idx2d0, insem0, asem0, wsem0),
        (srcb1, dstb1, efb1, s2d1, idx2d1, insem1, asem1, wsem1),
    )

    def _z(i, _):
        zbuf[pl.ds(i * 16, 16)] = jnp.zeros((16,), jnp.float32)
        return 0
    lax.fori_loop(0, STRIPE // 16, _z, 0)
    pltpu.sync_copy(zbuf, shared.at[pl.ds(sid * STRIPE, STRIPE)])

    pltpu.sync_copy(elr_hbm, elr_v)
    pltpu.sync_copy(ee_hbm, ee_v)
    plsc.subcore_barrier()

    iota4 = lax.iota(jnp.int32, 16) * 4

    def _load(k, b):
        base = (w + NW * k) * CHUNK
        pltpu.async_copy(ei_hbm.at[0, pl.ds(base, CHUNK)], b[0], b[5])
        pltpu.async_copy(ei_hbm.at[1, pl.ds(base, CHUNK)], b[1], b[5])
        pltpu.async_copy(ef_hbm.at[pl.ds(base, CHUNK)], b[2], b[5])

    def _wait_in(b):
        for r in (b[0], b[1], b[2]):
            pltpu.make_async_copy(ef_hbm.at[pl.ds(0, CHUNK)], r, b[5]).wait()

    def _compute(b):
        srcb, dstb, efb, s2d, idx2d = b[0], b[1], b[2], b[3], b[4]

        @plsc.parallel_loop(0, GROUPS, unroll=4)
        def _group(g):
            sv = srcb[pl.ds(g * 16, 16)]
            dv = dstb[pl.ds(g * 16, 16)]
            tv = efb[pl.ds(g * 16, 16)]
            bl = sv * 8
            br = dv * 8
            bt = tv * 4
            bd = dv * 4
            rowv = jnp.broadcast_to(g // 2, (16,)).astype(jnp.int32)
            cb = (g % 2) * 64
            for h in range(H):
                a = plsc.load_gather(elr_v, [bl + h])
                bb = plsc.load_gather(elr_v, [br + (4 + h)])
                c = plsc.load_gather(ee_v, [bt + h])
                s = jnp.exp(jnp.maximum(a + bb + c, 0.0))
                colv = iota4 + (cb + h)
                plsc.store_scatter(s2d, [rowv, colv], s)
                plsc.store_scatter(idx2d, [rowv, colv], bd + h)

    def _fire(k, b):
        s2d, idx2d, asem, wsem = b[3], b[4], b[6], b[7]
        ci = w + NW * k
        pltpu.async_copy(s2d, s_hbm.at[pl.ds(ci * SROWS, SROWS)], wsem)

        def _f(j, _):
            pltpu.async_copy(s2d.at[j], shared.at[idx2d.at[j]], asem,
                             add=True)
            return 0
        lax.fori_loop(0, SROWS, _f, 0)

    def _drain(b):
        s2d, idx2d, asem, wsem = b[3], b[4], b[6], b[7]
        pltpu.make_async_copy(s2d, s_hbm.at[pl.ds(0, SROWS)], wsem).wait()

        def _d(j, _):
            pltpu.make_async_copy(s2d.at[j], shared.at[idx2d.at[j]],
                                  asem).wait()
            return 0
        lax.fori_loop(0, SROWS, _d, 0)

    last = K - 1
    pred_last = (w + NW * last) < NCHUNKS

    _load(0, bufs[0])
    for k in range(last):
        b = bufs[k % 2]
        _wait_in(b)
        if k + 1 < last:
            _load(k + 1, bufs[(k + 1) % 2])
        else:
            pl.when(pred_last)(lambda: _load(last, bufs[last % 2]))
        if k >= 2:
            _drain(b)
        _compute(b)
        _fire(k, b)
    bl_ = bufs[last % 2]
    _drain(bl_)

    def _do_last():
        _wait_in(bl_)
        _compute(bl_)
        _fire(last, bl_)
    pl.when(pred_last)(_do_last)

    _drain(bufs[(last + 1) % 2])
    pl.when(pred_last)(lambda: _drain(bl_))

    plsc.subcore_barrier()

    pltpu.sync_copy(shared.at[pl.ds(sid * STRIPE, STRIPE)], zbuf)
    pltpu.sync_copy(zbuf,
                    dpart_hbm.at[pl.ds(cid * FLAT + sid * STRIPE, STRIPE)])


_sc_logits = functools.partial(
    pl.kernel,
    out_type=[
        jax.ShapeDtypeStruct((ROWS, 128), jnp.float32),
        jax.ShapeDtypeStruct((2 * FLAT,), jnp.float32),
    ],
    mesh=_mesh,
    compiler_params=_sc_params,
    scratch_types=[
        pltpu.VMEM((N * 2 * H,), jnp.float32),
        pltpu.VMEM((NET * H,), jnp.float32),
        pltpu.VMEM((CHUNK,), jnp.int32),
        pltpu.VMEM((CHUNK,), jnp.int32),
        pltpu.VMEM((CHUNK,), jnp.int32),
        pltpu.VMEM((SROWS, 128), jnp.float32),
        pltpu.VMEM((SROWS, 128), jnp.int32),
        pltpu.VMEM((CHUNK,), jnp.int32),
        pltpu.VMEM((CHUNK,), jnp.int32),
        pltpu.VMEM((CHUNK,), jnp.int32),
        pltpu.VMEM((SROWS, 128), jnp.float32),
        pltpu.VMEM((SROWS, 128), jnp.int32),
        pltpu.VMEM((STRIPE,), jnp.float32),
        pltpu.VMEM_SHARED((FLAT,), jnp.float32),
        pltpu.SemaphoreType.DMA,
        pltpu.SemaphoreType.DMA,
        pltpu.SemaphoreType.DMA,
        pltpu.SemaphoreType.DMA,
        pltpu.SemaphoreType.DMA,
        pltpu.SemaphoreType.DMA,
    ],
)(_sc_logits_body)


def _sc_div_body(s_hbm, ei_hbm, dpart_hbm, out_hbm,
                 dcomb, d1b,
                 s2d0, dstb0, o2d0, s2d1, dstb1, o2d1,
                 insem0, insem1, osem0, osem1):
    cid = lax.axis_index("c")
    sid = lax.axis_index("s")
    w = sid * 2 + cid

    bufs = (
        (s2d0, dstb0, o2d0, insem0, osem0),
        (s2d1, dstb1, o2d1, insem1, osem1),
    )

    pltpu.sync_copy(dpart_hbm.at[pl.ds(0, FLAT)], dcomb)
    pltpu.sync_copy(dpart_hbm.at[pl.ds(FLAT, FLAT)], d1b)

    iota1 = lax.iota(jnp.int32, 16)
    iota4 = iota1 * 4

    def _load(k, b):
        ci = w + NW * k
        pltpu.async_copy(s_hbm.at[pl.ds(ci * SROWS, SROWS)], b[0], b[3])
        pltpu.async_copy(ei_hbm.at[1, pl.ds(ci * CHUNK, CHUNK)], b[1], b[3])

    def _wait_in(b):
        pltpu.make_async_copy(s_hbm.at[pl.ds(0, SROWS)], b[0], b[3]).wait()
        pltpu.make_async_copy(ei_hbm.at[1, pl.ds(0, CHUNK)], b[1],
                              b[3]).wait()

    def _compute(b):
        s2d, dstb, o2d = b[0], b[1], b[2]

        @plsc.parallel_loop(0, GROUPS, unroll=4)
        def _group2(g):
            dv = dstb[pl.ds(g * 16, 16)]
            bd = dv * 4
            rowv = jnp.broadcast_to(g // 2, (16,)).astype(jnp.int32)
            cb = (g % 2) * 64
            tv = jnp.broadcast_to(g // 8, (16,)).astype(jnp.int32)
            lanev = iota1 + (g % 8) * 16
            for h in range(H):
                colv = iota4 + (cb + h)
                s = plsc.load_gather(s2d, [rowv, colv])
                d0 = plsc.load_gather(dcomb, [bd + h])
                d1 = plsc.load_gather(d1b, [bd + h])
                plsc.store_scatter(
                    o2d, [tv, jnp.full((16,), h, jnp.int32), lanev],
                    s / (d0 + d1))

    _T = CHUNK // 128

    def _fire(k, b):
        ci = w + NW * k
        pltpu.async_copy(b[2], out_hbm.at[pl.ds(ci * _T, _T)], b[4])

    def _drain(b):
        pltpu.make_async_copy(b[2], out_hbm.at[pl.ds(0, _T)], b[4]).wait()

    last = K - 1
    pred_last = (w + NW * last) < NCHUNKS

    _load(0, bufs[0])
    for k in range(last):
        b = bufs[k % 2]
        _wait_in(b)
        if k + 1 < last:
            _load(k + 1, bufs[(k + 1) % 2])
        else:
            pl.when(pred_last)(lambda: _load(last, bufs[last % 2]))
        if k >= 2:
            _drain(b)
        _compute(b)
        _fire(k, b)
    bl_ = bufs[last % 2]
    _drain(bl_)

    def _do_last():
        _wait_in(bl_)
        _compute(bl_)
        _fire(last, bl_)
    pl.when(pred_last)(_do_last)

    _drain(bufs[(last + 1) % 2])
    pl.when(pred_last)(lambda: _drain(bl_))


_sc_div = functools.partial(
    pl.kernel,
    out_type=jax.ShapeDtypeStruct((E // 128, H, 128), jnp.float32),
    mesh=_mesh,
    compiler_params=_sc_params_dense,
    scratch_types=[
        pltpu.VMEM((FLAT,), jnp.float32),
        pltpu.VMEM((FLAT,), jnp.float32),
        pltpu.VMEM((SROWS, 128), jnp.float32),
        pltpu.VMEM((CHUNK,), jnp.int32),
        pltpu.VMEM((CHUNK // 128, H, 128), jnp.float32),
        pltpu.VMEM((SROWS, 128), jnp.float32),
        pltpu.VMEM((CHUNK,), jnp.int32),
        pltpu.VMEM((CHUNK // 128, H, 128), jnp.float32),
        pltpu.SemaphoreType.DMA,
        pltpu.SemaphoreType.DMA,
        pltpu.SemaphoreType.DMA,
        pltpu.SemaphoreType.DMA,
    ],
)(_sc_div_body)


def kernel(x, edge_index, e_feat, edge_emb, W_fc, W_e, attn_l, attn_r, attn_e):
    elr, ee = _tc_project(x, W_fc, attn_l, attn_r, edge_emb, W_e, attn_e)
    s, dpart = _sc_logits(edge_index, e_feat, elr.reshape(-1), ee.reshape(-1))
    out3 = _sc_div(s, edge_index, dpart)
    return jnp.swapaxes(out3, 1, 2).reshape(E, H)


# speedup vs baseline: 1.0715x; 1.0715x over previous
import functools

import jax
import jax.numpy as jnp
from jax import lax
from jax.experimental import pallas as pl
from jax.experimental.pallas import tpu as pltpu
from jax.experimental.pallas import tpu_sc as plsc

N = 10000
E = 320000
D = 128
H = 4
EF = 16
NET = 64

NPAD = 10240
FLAT = NPAD * H
CHUNK = 1280
NW = 32
NCHUNKS = E // CHUNK
K = -(-NCHUNKS // NW)
GROUPS = CHUNK // 16
SROWS = CHUNK * H // 128
ROWS = E * H // 128
STRIPE = FLAT // 16


def _tc_body(x_ref, wfc_ref, al_ref, ar_ref, eemb_ref, we_ref, ae_ref,
             elr_ref, ee_ref):
    al = al_ref[0]
    ar = ar_ref[0]
    cols = []
    for a in (al, ar):
        for h in range(H):
            wh = wfc_ref[:, D * h:D * (h + 1)]
            cols.append(lax.dot_general(
                wh, a[h:h + 1, :], (((1,), (1,)), ((), ())),
                preferred_element_type=jnp.float32))
    v = jnp.concatenate(cols, axis=1)
    elr_ref[...] = lax.dot_general(
        x_ref[...], v, (((1,), (0,)), ((), ())),
        preferred_element_type=jnp.float32)
    proj = lax.dot_general(
        eemb_ref[...], we_ref[...], (((1,), (0,)), ((), ())),
        preferred_element_type=jnp.float32)
    ae = ae_ref[0]
    ecols = []
    for h in range(H):
        ecols.append(lax.dot_general(
            proj[:, EF * h:EF * (h + 1)], ae[h:h + 1, :],
            (((1,), (1,)), ((), ())),
            preferred_element_type=jnp.float32))
    ee_ref[...] = jnp.concatenate(ecols, axis=1)


_tc_project = pl.pallas_call(
    _tc_body,
    out_shape=[
        jax.ShapeDtypeStruct((N, 2 * H), jnp.float32),
        jax.ShapeDtypeStruct((NET, H), jnp.float32),
    ],
)


_mesh = plsc.VectorSubcoreMesh(core_axis_name="c", subcore_axis_name="s")
_sc_params = pltpu.CompilerParams(needs_layout_passes=False)
_sc_params_dense = pltpu.CompilerParams(
    needs_layout_passes=False, use_tc_tiling_on_sc=False)


def _sc_logits_body(ei_hbm, ef_hbm, elr_hbm, ee_hbm,
                    s_hbm, dpart_hbm,
                    elr_v, ee_v,
                    srcb0, dstb0, efb0, s2d0, idx2d0,
                    srcb1, dstb1, efb1, s2d1, idx2d1,
                    zbuf, shared,
                    insem0, insem1, asem0, asem1, wsem0, wsem1):
    cid = lax.axis_index("c")
    sid = lax.axis_index("s")
    w = sid * 2 + cid

    bufs = (
        (srcb0, dstb0, efb0, s2d0, idx2d0, insem0, asem0, wsem0),
        (srcb1, dstb1, efb1, s2d1, idx2d1, insem1, asem1, wsem1),
    )

    def _z(i, _):
        zbuf[pl.ds(i * 16, 16)] = jnp.zeros((16,), jnp.float32)
        return 0
    lax.fori_loop(0, STRIPE // 16, _z, 0)
    pltpu.sync_copy(zbuf, shared.at[pl.ds(sid * STRIPE, STRIPE)])

    pltpu.sync_copy(elr_hbm, elr_v)
    pltpu.sync_copy(ee_hbm, ee_v)
    plsc.subcore_barrier()

    iota4 = lax.iota(jnp.int32, 16) * 4

    def _load(k, b):
        base = (w + NW * k) * CHUNK
        pltpu.async_copy(ei_hbm.at[0, pl.ds(base, CHUNK)], b[0], b[5])
        pltpu.async_copy(ei_hbm.at[1, pl.ds(base, CHUNK)], b[1], b[5])
        pltpu.async_copy(ef_hbm.at[pl.ds(base, CHUNK)], b[2], b[5])

    def _wait_in(b):
        for r in (b[0], b[1], b[2]):
            pltpu.make_async_copy(ef_hbm.at[pl.ds(0, CHUNK)], r, b[5]).wait()

    def _compute(b):
        srcb, dstb, efb, s2d, idx2d = b[0], b[1], b[2], b[3], b[4]

        @plsc.parallel_loop(0, GROUPS, unroll=2)
        def _group(g):
            sv = srcb[pl.ds(g * 16, 16)]
            dv = dstb[pl.ds(g * 16, 16)]
            tv = efb[pl.ds(g * 16, 16)]
            bl = sv * 8
            br = dv * 8
            bt = tv * 4
            bd = dv * 4
            rowv = jnp.broadcast_to(g // 2, (16,)).astype(jnp.int32)
            cb = (g % 2) * 64
            for h in range(H):
                a = plsc.load_gather(elr_v, [bl + h])
                bb = plsc.load_gather(elr_v, [br + (4 + h)])
                c = plsc.load_gather(ee_v, [bt + h])
                s = jnp.exp(jnp.maximum(a + bb + c, 0.0))
                colv = iota4 + (cb + h)
                plsc.store_scatter(s2d, [rowv, colv], s)
                plsc.store_scatter(idx2d, [rowv, colv], bd + h)

    def _fire(k, b):
        s2d, idx2d, asem, wsem = b[3], b[4], b[6], b[7]
        ci = w + NW * k
        pltpu.async_copy(s2d, s_hbm.at[pl.ds(ci * SROWS, SROWS)], wsem)

        def _f(j, _):
            pltpu.async_copy(s2d.at[j], shared.at[idx2d.at[j]], asem,
                             add=True)
            return 0
        lax.fori_loop(0, SROWS, _f, 0)

    def _drain(b):
        s2d, idx2d, asem, wsem = b[3], b[4], b[6], b[7]
        pltpu.make_async_copy(s2d, s_hbm.at[pl.ds(0, SROWS)], wsem).wait()

        def _d(j, _):
            pltpu.make_async_copy(s2d.at[j], shared.at[idx2d.at[j]],
                                  asem).wait()
            return 0
        lax.fori_loop(0, SROWS, _d, 0)

    last = K - 1
    pred_last = (w + NW * last) < NCHUNKS

    _load(0, bufs[0])
    for k in range(last):
        b = bufs[k % 2]
        _wait_in(b)
        if k + 1 < last:
            _load(k + 1, bufs[(k + 1) % 2])
        else:
            pl.when(pred_last)(lambda: _load(last, bufs[last % 2]))
        if k >= 2:
            _drain(b)
        _compute(b)
        _fire(k, b)
    bl_ = bufs[last % 2]
    _drain(bl_)

    def _do_last():
        _wait_in(bl_)
        _compute(bl_)
        _fire(last, bl_)
    pl.when(pred_last)(_do_last)

    _drain(bufs[(last + 1) % 2])
    pl.when(pred_last)(lambda: _drain(bl_))

    plsc.subcore_barrier()

    pltpu.sync_copy(shared.at[pl.ds(sid * STRIPE, STRIPE)], zbuf)
    pltpu.sync_copy(zbuf,
                    dpart_hbm.at[pl.ds(cid * FLAT + sid * STRIPE, STRIPE)])


_sc_logits = functools.partial(
    pl.kernel,
    out_type=[
        jax.ShapeDtypeStruct((ROWS, 128), jnp.float32),
        jax.ShapeDtypeStruct((2 * FLAT,), jnp.float32),
    ],
    mesh=_mesh,
    compiler_params=_sc_params,
    scratch_types=[
        pltpu.VMEM((N * 2 * H,), jnp.float32),
        pltpu.VMEM((NET * H,), jnp.float32),
        pltpu.VMEM((CHUNK,), jnp.int32),
        pltpu.VMEM((CHUNK,), jnp.int32),
        pltpu.VMEM((CHUNK,), jnp.int32),
        pltpu.VMEM((SROWS, 128), jnp.float32),
        pltpu.VMEM((SROWS, 128), jnp.int32),
        pltpu.VMEM((CHUNK,), jnp.int32),
        pltpu.VMEM((CHUNK,), jnp.int32),
        pltpu.VMEM((CHUNK,), jnp.int32),
        pltpu.VMEM((SROWS, 128), jnp.float32),
        pltpu.VMEM((SROWS, 128), jnp.int32),
        pltpu.VMEM((STRIPE,), jnp.float32),
        pltpu.VMEM_SHARED((FLAT,), jnp.float32),
        pltpu.SemaphoreType.DMA,
        pltpu.SemaphoreType.DMA,
        pltpu.SemaphoreType.DMA,
        pltpu.SemaphoreType.DMA,
        pltpu.SemaphoreType.DMA,
        pltpu.SemaphoreType.DMA,
    ],
)(_sc_logits_body)


def _sc_div_body(s_hbm, ei_hbm, dpart_hbm, out_hbm,
                 dcomb, d1b,
                 s2d0, dstb0, o2d0, s2d1, dstb1, o2d1,
                 insem0, insem1, osem0, osem1):
    cid = lax.axis_index("c")
    sid = lax.axis_index("s")
    w = sid * 2 + cid

    bufs = (
        (s2d0, dstb0, o2d0, insem0, osem0),
        (s2d1, dstb1, o2d1, insem1, osem1),
    )

    pltpu.sync_copy(dpart_hbm.at[pl.ds(0, FLAT)], dcomb)
    pltpu.sync_copy(dpart_hbm.at[pl.ds(FLAT, FLAT)], d1b)

    iota1 = lax.iota(jnp.int32, 16)
    iota4 = iota1 * 4

    def _load(k, b):
        ci = w + NW * k
        pltpu.async_copy(s_hbm.at[pl.ds(ci * SROWS, SROWS)], b[0], b[3])
        pltpu.async_copy(ei_hbm.at[1, pl.ds(ci * CHUNK, CHUNK)], b[1], b[3])

    def _wait_in(b):
        pltpu.make_async_copy(s_hbm.at[pl.ds(0, SROWS)], b[0], b[3]).wait()
        pltpu.make_async_copy(ei_hbm.at[1, pl.ds(0, CHUNK)], b[1],
                              b[3]).wait()

    def _compute(b):
        s2d, dstb, o2d = b[0], b[1], b[2]

        @plsc.parallel_loop(0, GROUPS, unroll=2)
        def _group2(g):
            dv = dstb[pl.ds(g * 16, 16)]
            bd = dv * 4
            rowv = jnp.broadcast_to(g // 2, (16,)).astype(jnp.int32)
            cb = (g % 2) * 64
            tv = jnp.broadcast_to(g // 8, (16,)).astype(jnp.int32)
            lanev = iota1 + (g % 8) * 16
            for h in range(H):
                colv = iota4 + (cb + h)
                s = plsc.load_gather(s2d, [rowv, colv])
                d0 = plsc.load_gather(dcomb, [bd + h])
                d1 = plsc.load_gather(d1b, [bd + h])
                plsc.store_scatter(
                    o2d, [tv, jnp.full((16,), h, jnp.int32), lanev],
                    s / (d0 + d1))

    _T = CHUNK // 128

    def _fire(k, b):
        ci = w + NW * k
        pltpu.async_copy(b[2], out_hbm.at[pl.ds(ci * _T, _T)], b[4])

    def _drain(b):
        pltpu.make_async_copy(b[2], out_hbm.at[pl.ds(0, _T)], b[4]).wait()

    last = K - 1
    pred_last = (w + NW * last) < NCHUNKS

    _load(0, bufs[0])
    for k in range(last):
        b = bufs[k % 2]
        _wait_in(b)
        if k + 1 < last:
            _load(k + 1, bufs[(k + 1) % 2])
        else:
            pl.when(pred_last)(lambda: _load(last, bufs[last % 2]))
        if k >= 2:
            _drain(b)
        _compute(b)
        _fire(k, b)
    bl_ = bufs[last % 2]
    _drain(bl_)

    def _do_last():
        _wait_in(bl_)
        _compute(bl_)
        _fire(last, bl_)
    pl.when(pred_last)(_do_last)

    _drain(bufs[(last + 1) % 2])
    pl.when(pred_last)(lambda: _drain(bl_))


_sc_div = functools.partial(
    pl.kernel,
    out_type=jax.ShapeDtypeStruct((E // 128, H, 128), jnp.float32),
    mesh=_mesh,
    compiler_params=_sc_params_dense,
    scratch_types=[
        pltpu.VMEM((FLAT,), jnp.float32),
        pltpu.VMEM((FLAT,), jnp.float32),
        pltpu.VMEM((SROWS, 128), jnp.float32),
        pltpu.VMEM((CHUNK,), jnp.int32),
        pltpu.VMEM((CHUNK // 128, H, 128), jnp.float32),
        pltpu.VMEM((SROWS, 128), jnp.float32),
        pltpu.VMEM((CHUNK,), jnp.int32),
        pltpu.VMEM((CHUNK // 128, H, 128), jnp.float32),
        pltpu.SemaphoreType.DMA,
        pltpu.SemaphoreType.DMA,
        pltpu.SemaphoreType.DMA,
        pltpu.SemaphoreType.DMA,
    ],
)(_sc_div_body)


def kernel(x, edge_index, e_feat, edge_emb, W_fc, W_e, attn_l, attn_r, attn_e):
    elr, ee = _tc_project(x, W_fc, attn_l, attn_r, edge_emb, W_e, attn_e)
    s, dpart = _sc_logits(edge_index, e_feat, elr.reshape(-1), ee.reshape(-1))
    out3 = _sc_div(s, edge_index, dpart)
    return jnp.swapaxes(out3, 1, 2).reshape(E, H)
